# baseline (speedup 1.0000x reference)
import functools

import jax
import jax.numpy as jnp
from jax import lax
from jax.experimental import pallas as pl
from jax.experimental.pallas import tpu as pltpu
from jax.experimental.pallas import tpu_sc as plsc

N_TIME_INTERVAL = 100
PER_TIME = 10000.0
OUTPUT_DIM = 8

B = 4096
L = 200
TS_ROW = L + 1

NUM_CORES = 2
NUM_SUBCORES = 16
NW = NUM_CORES * NUM_SUBCORES
BPW = B // NW

L_CHUNK = 8
N_CHUNKS = L // L_CHUNK
NI = 8

_mesh = plsc.VectorSubcoreMesh(core_axis_name="c", subcore_axis_name="s")


@functools.partial(
    pl.kernel,
    out_type=(
        jax.ShapeDtypeStruct((L, NW, OUTPUT_DIM, BPW), jnp.float32),
        jax.ShapeDtypeStruct((L, B), jnp.int32),
    ),
    mesh=_mesh,
    scratch_types=[
        pltpu.VMEM((L, BPW), jnp.int32),
        pltpu.VMEM((BPW,), jnp.int32),
        pltpu.VMEM((2, L_CHUNK, OUTPUT_DIM, BPW), jnp.float32),
        pltpu.VMEM((N_TIME_INTERVAL * OUTPUT_DIM,), jnp.float32),
        pltpu.SemaphoreType.DMA((2,)),
        pltpu.SemaphoreType.DMA,
    ],
    compiler_params=pltpu.CompilerParams(
        needs_layout_passes=False, use_tc_tiling_on_sc=True),
)
def _time_encode(ts_hbm, ts_last_hbm, table_hbm, out_hbm, out2_hbm,
                 ts_v, ts_last_v, out_v, table_v, sem, sem2):
    wid = lax.axis_index("s") * NUM_CORES + lax.axis_index("c")
    b0 = wid * BPW
    pltpu.sync_copy(table_hbm, table_v)
    pltpu.sync_copy(ts_hbm.at[pl.ds(0, L), pl.ds(b0, BPW)], ts_v)
    pltpu.sync_copy(ts_last_hbm.at[pl.ds(b0, BPW)], ts_last_v)
    out2_cp = pltpu.async_copy(ts_v, out2_hbm.at[:, pl.ds(b0, BPW)], sem2)

    NG = BPW // 16

    def chunk(c, buf, last=False):
        l0 = c * L_CHUNK

        @pl.when(c >= 2)
        def _drain():
            pltpu.make_async_copy(
                out_v.at[buf], out_hbm.at[pl.ds(0, L_CHUNK), 0, :, :],
                sem.at[buf]).wait()

        def lane_pack(p, _):
            v16 = [(p * NI + g) * 16 for g in range(NI)]
            t_prev = [ts_v[l0, pl.ds(v16[g], 16)] for g in range(NI)]
            for lr in range(L_CHUNK):
                if last and lr == L_CHUNK - 1:
                    t_cur = [ts_last_v[pl.ds(v16[g], 16)] for g in range(NI)]
                else:
                    t_cur = [ts_v[l0 + lr + 1, pl.ds(v16[g], 16)]
                             for g in range(NI)]
                q = [(t_cur[g] - t_prev[g]).astype(jnp.float32) / PER_TIME
                     for g in range(NI)]
                idx = [q[g].astype(jnp.int32) for g in range(NI)]
                idx = [jnp.minimum(jnp.maximum(idx[g], 0), N_TIME_INTERVAL - 1)
                       for g in range(NI)]
                pos = [idx[g] * OUTPUT_DIM for g in range(NI)]
                prev = None
                for k in range(OUTPUT_DIM):
                    cur = []
                    for g in range(NI):
                        cur.append(plsc.load_gather(table_v, [pos[g] + k]))
                        if prev is not None:
                            out_v[buf, lr, k - 1, pl.ds(v16[g], 16)] = prev[g]
                    prev = cur
                for g in range(NI):
                    out_v[buf, lr, OUTPUT_DIM - 1, pl.ds(v16[g], 16)] = prev[g]
                    t_prev[g] = t_cur[g]
            return ()

        lax.fori_loop(0, NG // NI, lane_pack, (), unroll=1)

        pltpu.async_copy(
            out_v.at[buf], out_hbm.at[pl.ds(l0, L_CHUNK), wid, :, :],
            sem.at[buf])

    def chunk_pair(i, _):
        chunk(2 * i, 0)
        chunk(2 * i + 1, 1)
        return ()

    lax.fori_loop(0, (N_CHUNKS - 1) // 2, chunk_pair, (), unroll=1)
    chunk(N_CHUNKS - 1, 0, last=True)

    for buf in range(2):
        pltpu.make_async_copy(
            out_v.at[buf], out_hbm.at[pl.ds(0, L_CHUNK), 0, :, :],
            sem.at[buf]).wait()
    out2_cp.wait()


def kernel(input, timestamp, train, W, b):
    del input, train
    table = (W.T + b[None, :]).astype(jnp.float32).reshape(-1)
    tsi = timestamp.astype(jnp.int32)
    tst = tsi.T
    buf, buf2 = _time_encode(tst, tsi[:, L], table)
    out = buf.transpose(1, 3, 0, 2).reshape(B, L, OUTPUT_DIM)
    out2 = buf2.T.astype(timestamp.dtype)
    return (out, out2)


# speedup vs baseline: 1.3442x; 1.3442x over previous
import functools

import jax
import jax.numpy as jnp
from jax import lax
from jax.experimental import pallas as pl
from jax.experimental.pallas import tpu as pltpu
from jax.experimental.pallas import tpu_sc as plsc

N_TIME_INTERVAL = 100
PER_TIME = 10000.0
OUTPUT_DIM = 8

B = 4096
L = 200
TS_ROW = L + 1

NUM_CORES = 2
NUM_SUBCORES = 16
NW = NUM_CORES * NUM_SUBCORES
BPW = B // NW

L_CHUNK = 8
N_CHUNKS = L // L_CHUNK
NI = 4

_mesh = plsc.VectorSubcoreMesh(core_axis_name="c", subcore_axis_name="s")


@functools.partial(
    pl.kernel,
    out_type=(
        jax.ShapeDtypeStruct((L, NW, OUTPUT_DIM, BPW), jnp.float32),
        jax.ShapeDtypeStruct((L, B), jnp.int32),
    ),
    mesh=_mesh,
    scratch_types=[
        pltpu.VMEM((L, BPW), jnp.int32),
        pltpu.VMEM((BPW,), jnp.int32),
        pltpu.VMEM((2, L_CHUNK, OUTPUT_DIM, BPW), jnp.float32),
        pltpu.VMEM((N_TIME_INTERVAL * OUTPUT_DIM,), jnp.float32),
        pltpu.SemaphoreType.DMA((2,)),
        pltpu.SemaphoreType.DMA,
    ],
    compiler_params=pltpu.CompilerParams(
        needs_layout_passes=False, use_tc_tiling_on_sc=True),
)
def _time_encode(ts_hbm, ts_last_hbm, table_hbm, out_hbm, out2_hbm,
                 ts_v, ts_last_v, out_v, table_v, sem, sem2):
    wid = lax.axis_index("s") * NUM_CORES + lax.axis_index("c")
    b0 = wid * BPW
    pltpu.sync_copy(table_hbm, table_v)
    pltpu.sync_copy(ts_hbm.at[pl.ds(0, L), pl.ds(b0, BPW)], ts_v)
    pltpu.sync_copy(ts_last_hbm.at[pl.ds(b0, BPW)], ts_last_v)
    out2_cp = pltpu.async_copy(ts_v, out2_hbm.at[:, pl.ds(b0, BPW)], sem2)

    NG = BPW // 16

    def chunk(c, buf, last=False):
        l0 = c * L_CHUNK

        @pl.when(c >= 2)
        def _drain():
            pltpu.make_async_copy(
                out_v.at[buf], out_hbm.at[pl.ds(0, L_CHUNK), 0, :, :],
                sem.at[buf]).wait()

        def lane_pack(p, _):
            v16 = [(p * NI + g) * 16 for g in range(NI)]
            t_prev = [ts_v[l0, pl.ds(v16[g], 16)] for g in range(NI)]
            for lr in range(L_CHUNK):
                if last and lr == L_CHUNK - 1:
                    t_cur = [ts_last_v[pl.ds(v16[g], 16)] for g in range(NI)]
                else:
                    t_cur = [ts_v[l0 + lr + 1, pl.ds(v16[g], 16)]
                             for g in range(NI)]
                q = [(t_cur[g] - t_prev[g]).astype(jnp.float32) / PER_TIME
                     for g in range(NI)]
                idx = [q[g].astype(jnp.int32) for g in range(NI)]
                idx = [jnp.minimum(jnp.maximum(idx[g], 0), N_TIME_INTERVAL - 1)
                       for g in range(NI)]
                pos = [idx[g] * OUTPUT_DIM for g in range(NI)]
                prev = None
                for k in range(OUTPUT_DIM):
                    cur = []
                    for g in range(NI):
                        cur.append(plsc.load_gather(table_v, [pos[g] + k]))
                        if prev is not None:
                            out_v[buf, lr, k - 1, pl.ds(v16[g], 16)] = prev[g]
                    prev = cur
                for g in range(NI):
                    out_v[buf, lr, OUTPUT_DIM - 1, pl.ds(v16[g], 16)] = prev[g]
                    t_prev[g] = t_cur[g]
            return ()

        lax.fori_loop(0, NG // NI, lane_pack, (), unroll=1)

        pltpu.async_copy(
            out_v.at[buf], out_hbm.at[pl.ds(l0, L_CHUNK), wid, :, :],
            sem.at[buf])

    def chunk_pair(i, _):
        chunk(2 * i, 0)
        chunk(2 * i + 1, 1)
        return ()

    lax.fori_loop(0, (N_CHUNKS - 1) // 2, chunk_pair, (), unroll=1)
    chunk(N_CHUNKS - 1, 0, last=True)

    for buf in range(2):
        pltpu.make_async_copy(
            out_v.at[buf], out_hbm.at[pl.ds(0, L_CHUNK), 0, :, :],
            sem.at[buf]).wait()
    out2_cp.wait()


def kernel(input, timestamp, train, W, b):
    del input, train
    table = (W.T + b[None, :]).astype(jnp.float32).reshape(-1)
    tsi = timestamp.astype(jnp.int32)
    tst = tsi.T
    buf, buf2 = _time_encode(tst, tsi[:, L], table)
    out = buf.transpose(1, 3, 0, 2).reshape(B, L, OUTPUT_DIM)
    out2 = buf2.T.astype(timestamp.dtype)
    return (out, out2)
